# baseline (speedup 1.0000x reference)
import functools

import jax
import jax.numpy as jnp
from jax import lax
from jax.experimental import pallas as pl
from jax.experimental.pallas import tpu as pltpu
from jax.experimental.pallas import tpu_sc as plsc

N = 10000
E = 320000
D_IN = 128
DH = 64
G = 64

NC = 2
NS = 16
NW = NC * NS
L = 16

RB = 400
NB = N // RB
SL = N // NS
ZR = SL // 5
NPADH = 10240
SLH = NPADH // NS
CH = 128
NCHK = E // CH
KMAX = NCHK // NW + 1
NBIG = NW - (KMAX * NW - NCHK)

_mesh = plsc.VectorSubcoreMesh(core_axis_name="c", subcore_axis_name="s")
_sc_params = pltpu.CompilerParams(needs_layout_passes=False,
                                  use_tc_tiling_on_sc=False)


def _tile_range(wid):
    base = (KMAX - 1) * wid + jnp.minimum(wid, NBIG)
    cnt = jnp.where(wid < NBIG, KMAX, KMAX - 1)
    return base, cnt


@functools.partial(
    pl.kernel,
    out_type=jax.ShapeDtypeStruct((NC, NPADH), jnp.float32),
    mesh=_mesh,
    compiler_params=_sc_params,
    scratch_types=[
        pltpu.VMEM((KMAX * CH,), jnp.int32),
        pltpu.VMEM((NPADH,), jnp.float32),
        pltpu.VMEM((SLH,), jnp.float32),
        pltpu.VMEM((SLH,), jnp.float32),
        pltpu.VMEM_SHARED((NS, NPADH), jnp.float32),
    ],
)
def _sc_degree(dst_hbm, deg_out, idx_v, hist, acc_v, tmp_v, hist_sh):
    cid = lax.axis_index("c")
    sid = lax.axis_index("s")
    wid = cid * NS + sid
    base, cnt = _tile_range(wid)

    @pl.when(cnt == KMAX)
    def _():
        pltpu.sync_copy(dst_hbm.at[pl.ds(base * CH, KMAX * CH)], idx_v)

    @pl.when(cnt != KMAX)
    def _():
        pltpu.sync_copy(dst_hbm.at[pl.ds(base * CH, (KMAX - 1) * CH)],
                        idx_v.at[pl.ds(0, (KMAX - 1) * CH)])

    def zb(i, _):
        hist[pl.ds(i * L, L)] = jnp.zeros((L,), jnp.float32)
        return 0

    lax.fori_loop(0, NPADH // L, zb, 0)

    ones = jnp.ones((L,), jnp.float32)

    def eb(i, _):
        ids = idx_v[pl.ds(i * L, L)]
        plsc.addupdate_scatter(hist, [ids], ones)
        return 0

    lax.fori_loop(0, cnt * (CH // L), eb, 0)

    pltpu.sync_copy(hist, hist_sh.at[sid])
    plsc.subcore_barrier()

    def za(i, _):
        acc_v[pl.ds(i * L, L)] = jnp.zeros((L,), jnp.float32)
        return 0

    lax.fori_loop(0, SLH // L, za, 0)
    for h in range(NS):
        pltpu.sync_copy(hist_sh.at[h, pl.ds(sid * SLH, SLH)], tmp_v)

        def ab(i, _):
            sl = pl.ds(i * L, L)
            acc_v[sl] = acc_v[sl] + tmp_v[sl]
            return 0

        lax.fori_loop(0, SLH // L, ab, 0)
    pltpu.sync_copy(acc_v, deg_out.at[cid, pl.ds(sid * SLH, SLH)])


@functools.partial(
    pl.kernel,
    out_type=jax.ShapeDtypeStruct((NC, N, DH), jnp.float32),
    mesh=_mesh,
    compiler_params=_sc_params,
    scratch_types=[
        pltpu.VMEM((KMAX, CH), jnp.int32),
        pltpu.VMEM((KMAX, CH), jnp.int32),
        pltpu.VMEM((CH, DH), jnp.float32),
        pltpu.VMEM((CH, DH), jnp.float32),
        pltpu.VMEM((ZR, DH), jnp.float32),
        pltpu.VMEM_SHARED((N, DH), jnp.float32),
        pltpu.SemaphoreType.DMA,
        pltpu.SemaphoreType.DMA,
    ],
)
def _sc_edge_pass(h_hbm, src_hbm, dst_hbm, out_hbm,
                  src_v, dst_v, rows0, rows1, zbuf, acc_sh, sem0, sem1):
    cid = lax.axis_index("c")
    sid = lax.axis_index("s")
    wid = cid * NS + sid
    base, cnt = _tile_range(wid)

    @pl.when(cnt == KMAX)
    def _():
        pltpu.sync_copy(src_hbm.at[pl.ds(base, KMAX)], src_v)
        pltpu.sync_copy(dst_hbm.at[pl.ds(base, KMAX)], dst_v)

    @pl.when(cnt != KMAX)
    def _():
        pltpu.sync_copy(src_hbm.at[pl.ds(base, KMAX - 1)],
                        src_v.at[pl.ds(0, KMAX - 1)])
        pltpu.sync_copy(dst_hbm.at[pl.ds(base, KMAX - 1)],
                        dst_v.at[pl.ds(0, KMAX - 1)])

    def zv(i, _):
        zbuf[i, pl.ds(0, L)] = jnp.zeros((L,), jnp.float32)
        zbuf[i, pl.ds(L, L)] = jnp.zeros((L,), jnp.float32)
        zbuf[i, pl.ds(2 * L, L)] = jnp.zeros((L,), jnp.float32)
        zbuf[i, pl.ds(3 * L, L)] = jnp.zeros((L,), jnp.float32)
        return 0

    lax.fori_loop(0, ZR, zv, 0)
    for z in range(SL // ZR):
        pltpu.sync_copy(zbuf, acc_sh.at[pl.ds(sid * SL + z * ZR, ZR)])
    plsc.subcore_barrier()

    pltpu.async_copy(h_hbm.at[src_v.at[0]], rows0, sem0)
    pltpu.async_copy(h_hbm.at[src_v.at[1]], rows1, sem1)

    def body(t, _):
        j0 = 2 * t
        j1 = j0 + 1
        pltpu.make_async_copy(h_hbm.at[src_v.at[j0]], rows0, sem0).wait()
        pltpu.sync_copy(rows0, acc_sh.at[dst_v.at[j0]], add=True)

        @pl.when(j0 + 2 < cnt)
        def _():
            pltpu.async_copy(h_hbm.at[src_v.at[j0 + 2]], rows0, sem0)

        pltpu.make_async_copy(h_hbm.at[src_v.at[j1]], rows1, sem1).wait()
        pltpu.sync_copy(rows1, acc_sh.at[dst_v.at[j1]], add=True)

        @pl.when(j1 + 2 < cnt)
        def _():
            pltpu.async_copy(h_hbm.at[src_v.at[j1 + 2]], rows1, sem1)

        return 0

    lax.fori_loop(0, cnt // 2, body, 0)

    @pl.when(cnt % 2 == 1)
    def _():
        j = cnt - 1
        pltpu.make_async_copy(h_hbm.at[src_v.at[j]], rows0, sem0).wait()
        pltpu.sync_copy(rows0, acc_sh.at[dst_v.at[j]], add=True)

    plsc.subcore_barrier()
    nsl = pl.ds(sid * SL, SL)
    pltpu.sync_copy(acc_sh.at[nsl], out_hbm.at[cid, nsl])


def _tc_a_body(x_ref, w1_ref, d0_ref, d1_ref, h_ref, dis_ref):
    deg = d0_ref[...] + d1_ref[...] + 1.0
    dis = lax.rsqrt(deg)
    h = jnp.dot(x_ref[...], w1_ref[...], preferred_element_type=jnp.float32)
    h_ref[...] = h * dis
    dis_ref[...] = dis


def _tc_a(x, W1, deg0, deg1):
    return pl.pallas_call(
        _tc_a_body,
        grid=(NB,),
        in_specs=[
            pl.BlockSpec((RB, D_IN), lambda i: (i, 0)),
            pl.BlockSpec((D_IN, DH), lambda i: (0, 0)),
            pl.BlockSpec((RB, 1), lambda i: (i, 0)),
            pl.BlockSpec((RB, 1), lambda i: (i, 0)),
        ],
        out_specs=[
            pl.BlockSpec((RB, DH), lambda i: (i, 0)),
            pl.BlockSpec((RB, 1), lambda i: (i, 0)),
        ],
        out_shape=[
            jax.ShapeDtypeStruct((N, DH), jnp.float32),
            jax.ShapeDtypeStruct((N, 1), jnp.float32),
        ],
    )(x, W1, deg0, deg1)


def _tc_b_body(s0_ref, s1_ref, hp_ref, dis_ref, w2_ref, b1_ref, out_ref):
    dis = dis_ref[...]
    z = jax.nn.relu(dis * (s0_ref[...] + s1_ref[...] + hp_ref[...])
                    + b1_ref[...])
    out_ref[...] = jnp.dot(z, w2_ref[...],
                           preferred_element_type=jnp.float32) * dis


def _tc_b(s0, s1, h1p, dis, W2, b1r):
    return pl.pallas_call(
        _tc_b_body,
        grid=(NB,),
        in_specs=[
            pl.BlockSpec((RB, DH), lambda i: (i, 0)),
            pl.BlockSpec((RB, DH), lambda i: (i, 0)),
            pl.BlockSpec((RB, DH), lambda i: (i, 0)),
            pl.BlockSpec((RB, 1), lambda i: (i, 0)),
            pl.BlockSpec((DH, DH), lambda i: (0, 0)),
            pl.BlockSpec((1, DH), lambda i: (0, 0)),
        ],
        out_specs=pl.BlockSpec((RB, DH), lambda i: (i, 0)),
        out_shape=jax.ShapeDtypeStruct((N, DH), jnp.float32),
    )(s0, s1, h1p, dis, W2, b1r)


def _tc_c_body(s0_ref, s1_ref, hp_ref, dis_ref, b2_ref, bt_ref,
               wf1_ref, bf1_ref, wf2_ref, bf2_ref, out_ref,
               sum_acc, max_acc, cnt_acc):
    pid = pl.program_id(0)

    @pl.when(pid == 0)
    def _():
        sum_acc[...] = jnp.zeros((G, DH), jnp.float32)
        max_acc[...] = jnp.full((G, DH), -jnp.inf, jnp.float32)
        cnt_acc[...] = jnp.zeros((G, 1), jnp.float32)

    h2 = (dis_ref[...] * (s0_ref[...] + s1_ref[...] + hp_ref[...])
          + b2_ref[...])
    bt = bt_ref[...]
    gi = lax.broadcasted_iota(jnp.int32, (RB, G), 1)
    onehot = jnp.where(bt == gi, 1.0, 0.0)
    sum_acc[...] += lax.dot_general(
        onehot, h2, (((0,), (0,)), ((), ())),
        preferred_element_type=jnp.float32)
    cnt_acc[...] += lax.dot_general(
        onehot, jnp.ones((RB, 1), jnp.float32), (((0,), (0,)), ((), ())),
        preferred_element_type=jnp.float32)

    glo = jnp.min(bt)
    ghi = jnp.max(bt)

    def gbody(g, _):
        m = jnp.where(bt == g, h2, -jnp.inf)
        colmax = jnp.max(m, axis=0, keepdims=True)
        sl = pl.ds(g, 1)
        max_acc[sl, :] = jnp.maximum(max_acc[sl, :], colmax)
        return 0

    lax.fori_loop(glo, ghi + 1, gbody, 0)

    @pl.when(pid == NB - 1)
    def _():
        mean = sum_acc[...] / jnp.maximum(cnt_acc[...], 1.0)
        pooled = jnp.concatenate([mean, max_acc[...]], axis=1)
        z = jax.nn.relu(
            jnp.dot(pooled, wf1_ref[...], preferred_element_type=jnp.float32)
            + bf1_ref[...])
        o = jnp.dot(z, wf2_ref[...], preferred_element_type=jnp.float32)
        out_ref[...] = o[:, 0:1] + bf2_ref[...]


def _tc_c(s0, s1, h2p, dis, b2r, batch_c, Wfc1, bfc1r, Wfc2p, bfc2r):
    return pl.pallas_call(
        _tc_c_body,
        grid=(NB,),
        in_specs=[
            pl.BlockSpec((RB, DH), lambda i: (i, 0)),
            pl.BlockSpec((RB, DH), lambda i: (i, 0)),
            pl.BlockSpec((RB, DH), lambda i: (i, 0)),
            pl.BlockSpec((RB, 1), lambda i: (i, 0)),
            pl.BlockSpec((1, DH), lambda i: (0, 0)),
            pl.BlockSpec((RB, 1), lambda i: (i, 0)),
            pl.BlockSpec((2 * DH, DH), lambda i: (0, 0)),
            pl.BlockSpec((1, DH), lambda i: (0, 0)),
            pl.BlockSpec((DH, 128), lambda i: (0, 0)),
            pl.BlockSpec((1, 1), lambda i: (0, 0)),
        ],
        out_specs=pl.BlockSpec((G, 1), lambda i: (0, 0)),
        out_shape=jax.ShapeDtypeStruct((G, 1), jnp.float32),
        scratch_shapes=[
            pltpu.VMEM((G, DH), jnp.float32),
            pltpu.VMEM((G, DH), jnp.float32),
            pltpu.VMEM((G, 1), jnp.float32),
        ],
    )(s0, s1, h2p, dis, b2r, batch_c, Wfc1, bfc1r, Wfc2p, bfc2r)


@jax.jit
def kernel(x, edge_index, batch, W1, b1, W2, b2, Wfc1, bfc1, Wfc2, bfc2):
    f32 = jnp.float32
    src3 = edge_index[0].astype(jnp.int32).reshape(NCHK, CH)
    dst1 = edge_index[1].astype(jnp.int32)
    dst3 = dst1.reshape(NCHK, CH)
    batch_c = batch.astype(jnp.int32).reshape(N, 1)

    deg_p = _sc_degree(dst1)
    deg0 = deg_p[0, :N].reshape(N, 1)
    deg1 = deg_p[1, :N].reshape(N, 1)

    h1p, dis = _tc_a(x, W1, deg0, deg1)
    s1 = _sc_edge_pass(h1p, src3, dst3)
    h2p = _tc_b(s1[0], s1[1], h1p, dis, W2, b1.reshape(1, DH))
    s2 = _sc_edge_pass(h2p, src3, dst3)

    Wfc2p = jnp.zeros((DH, 128), f32).at[:, 0:1].set(Wfc2)
    out = _tc_c(s2[0], s2[1], h2p, dis, b2.reshape(1, DH), batch_c,
                Wfc1, bfc1.reshape(1, DH), Wfc2p, bfc2.reshape(1, 1))
    return out


# speedup vs baseline: 1.0004x; 1.0004x over previous
import functools

import jax
import jax.numpy as jnp
from jax import lax
from jax.experimental import pallas as pl
from jax.experimental.pallas import tpu as pltpu
from jax.experimental.pallas import tpu_sc as plsc

N = 10000
E = 320000
D_IN = 128
DH = 64
G = 64

NC = 2
NS = 16
NW = NC * NS
L = 16

RB = 400
NB = N // RB
SL = N // NS
ZR = SL // 5
NPADH = 10240
SLH = NPADH // NS
CH = 128
NCHK = E // CH
KMAX = NCHK // NW + 1
NBIG = NW - (KMAX * NW - NCHK)

_mesh = plsc.VectorSubcoreMesh(core_axis_name="c", subcore_axis_name="s")
_sc_params = pltpu.CompilerParams(needs_layout_passes=False,
                                  use_tc_tiling_on_sc=False,
                                  skip_device_barrier=True)


def _tile_range(wid):
    base = (KMAX - 1) * wid + jnp.minimum(wid, NBIG)
    cnt = jnp.where(wid < NBIG, KMAX, KMAX - 1)
    return base, cnt


@functools.partial(
    pl.kernel,
    out_type=jax.ShapeDtypeStruct((NC, NPADH), jnp.float32),
    mesh=_mesh,
    compiler_params=_sc_params,
    scratch_types=[
        pltpu.VMEM((KMAX * CH,), jnp.int32),
        pltpu.VMEM((NPADH,), jnp.float32),
        pltpu.VMEM((SLH,), jnp.float32),
        pltpu.VMEM((SLH,), jnp.float32),
        pltpu.VMEM_SHARED((NS, NPADH), jnp.float32),
    ],
)
def _sc_degree(dst_hbm, deg_out, idx_v, hist, acc_v, tmp_v, hist_sh):
    cid = lax.axis_index("c")
    sid = lax.axis_index("s")
    wid = cid * NS + sid
    base, cnt = _tile_range(wid)

    @pl.when(cnt == KMAX)
    def _():
        pltpu.sync_copy(dst_hbm.at[pl.ds(base * CH, KMAX * CH)], idx_v)

    @pl.when(cnt != KMAX)
    def _():
        pltpu.sync_copy(dst_hbm.at[pl.ds(base * CH, (KMAX - 1) * CH)],
                        idx_v.at[pl.ds(0, (KMAX - 1) * CH)])

    def zb(i, _):
        hist[pl.ds(i * L, L)] = jnp.zeros((L,), jnp.float32)
        return 0

    lax.fori_loop(0, NPADH // L, zb, 0)

    ones = jnp.ones((L,), jnp.float32)

    def eb(i, _):
        ids = idx_v[pl.ds(i * L, L)]
        plsc.addupdate_scatter(hist, [ids], ones)
        return 0

    lax.fori_loop(0, cnt * (CH // L), eb, 0)

    pltpu.sync_copy(hist, hist_sh.at[sid])
    plsc.subcore_barrier()

    def za(i, _):
        acc_v[pl.ds(i * L, L)] = jnp.zeros((L,), jnp.float32)
        return 0

    lax.fori_loop(0, SLH // L, za, 0)
    for h in range(NS):
        pltpu.sync_copy(hist_sh.at[h, pl.ds(sid * SLH, SLH)], tmp_v)

        def ab(i, _):
            sl = pl.ds(i * L, L)
            acc_v[sl] = acc_v[sl] + tmp_v[sl]
            return 0

        lax.fori_loop(0, SLH // L, ab, 0)
    pltpu.sync_copy(acc_v, deg_out.at[cid, pl.ds(sid * SLH, SLH)])


@functools.partial(
    pl.kernel,
    out_type=jax.ShapeDtypeStruct((NC, N, DH), jnp.float32),
    mesh=_mesh,
    compiler_params=_sc_params,
    scratch_types=[
        pltpu.VMEM((KMAX, CH), jnp.int32),
        pltpu.VMEM((KMAX, CH), jnp.int32),
        pltpu.VMEM((CH, DH), jnp.float32),
        pltpu.VMEM((CH, DH), jnp.float32),
        pltpu.VMEM((ZR, DH), jnp.float32),
        pltpu.VMEM_SHARED((N, DH), jnp.float32),
        pltpu.SemaphoreType.DMA,
        pltpu.SemaphoreType.DMA,
    ],
)
def _sc_edge_pass(h_hbm, src_hbm, dst_hbm, out_hbm,
                  src_v, dst_v, rows0, rows1, zbuf, acc_sh, sem0, sem1):
    cid = lax.axis_index("c")
    sid = lax.axis_index("s")
    wid = cid * NS + sid
    base, cnt = _tile_range(wid)

    @pl.when(cnt == KMAX)
    def _():
        pltpu.sync_copy(src_hbm.at[pl.ds(base, KMAX)], src_v)
        pltpu.sync_copy(dst_hbm.at[pl.ds(base, KMAX)], dst_v)

    @pl.when(cnt != KMAX)
    def _():
        pltpu.sync_copy(src_hbm.at[pl.ds(base, KMAX - 1)],
                        src_v.at[pl.ds(0, KMAX - 1)])
        pltpu.sync_copy(dst_hbm.at[pl.ds(base, KMAX - 1)],
                        dst_v.at[pl.ds(0, KMAX - 1)])

    def zv(i, _):
        zbuf[i, pl.ds(0, L)] = jnp.zeros((L,), jnp.float32)
        zbuf[i, pl.ds(L, L)] = jnp.zeros((L,), jnp.float32)
        zbuf[i, pl.ds(2 * L, L)] = jnp.zeros((L,), jnp.float32)
        zbuf[i, pl.ds(3 * L, L)] = jnp.zeros((L,), jnp.float32)
        return 0

    lax.fori_loop(0, ZR, zv, 0)
    for z in range(SL // ZR):
        pltpu.sync_copy(zbuf, acc_sh.at[pl.ds(sid * SL + z * ZR, ZR)])
    plsc.subcore_barrier()

    pltpu.async_copy(h_hbm.at[src_v.at[0]], rows0, sem0)
    pltpu.async_copy(h_hbm.at[src_v.at[1]], rows1, sem1)

    def body(t, _):
        j0 = 2 * t
        j1 = j0 + 1
        pltpu.make_async_copy(h_hbm.at[src_v.at[j0]], rows0, sem0).wait()
        pltpu.sync_copy(rows0, acc_sh.at[dst_v.at[j0]], add=True)

        @pl.when(j0 + 2 < cnt)
        def _():
            pltpu.async_copy(h_hbm.at[src_v.at[j0 + 2]], rows0, sem0)

        pltpu.make_async_copy(h_hbm.at[src_v.at[j1]], rows1, sem1).wait()
        pltpu.sync_copy(rows1, acc_sh.at[dst_v.at[j1]], add=True)

        @pl.when(j1 + 2 < cnt)
        def _():
            pltpu.async_copy(h_hbm.at[src_v.at[j1 + 2]], rows1, sem1)

        return 0

    lax.fori_loop(0, cnt // 2, body, 0)

    @pl.when(cnt % 2 == 1)
    def _():
        j = cnt - 1
        pltpu.make_async_copy(h_hbm.at[src_v.at[j]], rows0, sem0).wait()
        pltpu.sync_copy(rows0, acc_sh.at[dst_v.at[j]], add=True)

    plsc.subcore_barrier()
    nsl = pl.ds(sid * SL, SL)
    pltpu.sync_copy(acc_sh.at[nsl], out_hbm.at[cid, nsl])


def _tc_a_body(x_ref, w1_ref, d0_ref, d1_ref, h_ref, dis_ref):
    deg = d0_ref[...] + d1_ref[...] + 1.0
    dis = lax.rsqrt(deg)
    h = jnp.dot(x_ref[...], w1_ref[...], preferred_element_type=jnp.float32)
    h_ref[...] = h * dis
    dis_ref[...] = dis


def _tc_a(x, W1, deg0, deg1):
    return pl.pallas_call(
        _tc_a_body,
        grid=(NB,),
        in_specs=[
            pl.BlockSpec((RB, D_IN), lambda i: (i, 0)),
            pl.BlockSpec((D_IN, DH), lambda i: (0, 0)),
            pl.BlockSpec((RB, 1), lambda i: (i, 0)),
            pl.BlockSpec((RB, 1), lambda i: (i, 0)),
        ],
        out_specs=[
            pl.BlockSpec((RB, DH), lambda i: (i, 0)),
            pl.BlockSpec((RB, 1), lambda i: (i, 0)),
        ],
        out_shape=[
            jax.ShapeDtypeStruct((N, DH), jnp.float32),
            jax.ShapeDtypeStruct((N, 1), jnp.float32),
        ],
    )(x, W1, deg0, deg1)


def _tc_b_body(s0_ref, s1_ref, hp_ref, dis_ref, w2_ref, b1_ref, out_ref):
    dis = dis_ref[...]
    z = jax.nn.relu(dis * (s0_ref[...] + s1_ref[...] + hp_ref[...])
                    + b1_ref[...])
    out_ref[...] = jnp.dot(z, w2_ref[...],
                           preferred_element_type=jnp.float32) * dis


def _tc_b(s0, s1, h1p, dis, W2, b1r):
    return pl.pallas_call(
        _tc_b_body,
        grid=(NB,),
        in_specs=[
            pl.BlockSpec((RB, DH), lambda i: (i, 0)),
            pl.BlockSpec((RB, DH), lambda i: (i, 0)),
            pl.BlockSpec((RB, DH), lambda i: (i, 0)),
            pl.BlockSpec((RB, 1), lambda i: (i, 0)),
            pl.BlockSpec((DH, DH), lambda i: (0, 0)),
            pl.BlockSpec((1, DH), lambda i: (0, 0)),
        ],
        out_specs=pl.BlockSpec((RB, DH), lambda i: (i, 0)),
        out_shape=jax.ShapeDtypeStruct((N, DH), jnp.float32),
    )(s0, s1, h1p, dis, W2, b1r)


def _tc_c_body(s0_ref, s1_ref, hp_ref, dis_ref, b2_ref, bt_ref,
               wf1_ref, bf1_ref, wf2_ref, bf2_ref, out_ref,
               sum_acc, max_acc, cnt_acc):
    pid = pl.program_id(0)

    @pl.when(pid == 0)
    def _():
        sum_acc[...] = jnp.zeros((G, DH), jnp.float32)
        max_acc[...] = jnp.full((G, DH), -jnp.inf, jnp.float32)
        cnt_acc[...] = jnp.zeros((G, 1), jnp.float32)

    h2 = (dis_ref[...] * (s0_ref[...] + s1_ref[...] + hp_ref[...])
          + b2_ref[...])
    bt = bt_ref[...]
    gi = lax.broadcasted_iota(jnp.int32, (RB, G), 1)
    onehot = jnp.where(bt == gi, 1.0, 0.0)
    sum_acc[...] += lax.dot_general(
        onehot, h2, (((0,), (0,)), ((), ())),
        preferred_element_type=jnp.float32)
    cnt_acc[...] += lax.dot_general(
        onehot, jnp.ones((RB, 1), jnp.float32), (((0,), (0,)), ((), ())),
        preferred_element_type=jnp.float32)

    glo = jnp.min(bt)
    ghi = jnp.max(bt)

    def gbody(g, _):
        m = jnp.where(bt == g, h2, -jnp.inf)
        colmax = jnp.max(m, axis=0, keepdims=True)
        sl = pl.ds(g, 1)
        max_acc[sl, :] = jnp.maximum(max_acc[sl, :], colmax)
        return 0

    lax.fori_loop(glo, ghi + 1, gbody, 0)

    @pl.when(pid == NB - 1)
    def _():
        mean = sum_acc[...] / jnp.maximum(cnt_acc[...], 1.0)
        pooled = jnp.concatenate([mean, max_acc[...]], axis=1)
        z = jax.nn.relu(
            jnp.dot(pooled, wf1_ref[...], preferred_element_type=jnp.float32)
            + bf1_ref[...])
        o = jnp.dot(z, wf2_ref[...], preferred_element_type=jnp.float32)
        out_ref[...] = o[:, 0:1] + bf2_ref[...]


def _tc_c(s0, s1, h2p, dis, b2r, batch_c, Wfc1, bfc1r, Wfc2p, bfc2r):
    return pl.pallas_call(
        _tc_c_body,
        grid=(NB,),
        in_specs=[
            pl.BlockSpec((RB, DH), lambda i: (i, 0)),
            pl.BlockSpec((RB, DH), lambda i: (i, 0)),
            pl.BlockSpec((RB, DH), lambda i: (i, 0)),
            pl.BlockSpec((RB, 1), lambda i: (i, 0)),
            pl.BlockSpec((1, DH), lambda i: (0, 0)),
            pl.BlockSpec((RB, 1), lambda i: (i, 0)),
            pl.BlockSpec((2 * DH, DH), lambda i: (0, 0)),
            pl.BlockSpec((1, DH), lambda i: (0, 0)),
            pl.BlockSpec((DH, 128), lambda i: (0, 0)),
            pl.BlockSpec((1, 1), lambda i: (0, 0)),
        ],
        out_specs=pl.BlockSpec((G, 1), lambda i: (0, 0)),
        out_shape=jax.ShapeDtypeStruct((G, 1), jnp.float32),
        scratch_shapes=[
            pltpu.VMEM((G, DH), jnp.float32),
            pltpu.VMEM((G, DH), jnp.float32),
            pltpu.VMEM((G, 1), jnp.float32),
        ],
    )(s0, s1, h2p, dis, b2r, batch_c, Wfc1, bfc1r, Wfc2p, bfc2r)


@jax.jit
def kernel(x, edge_index, batch, W1, b1, W2, b2, Wfc1, bfc1, Wfc2, bfc2):
    f32 = jnp.float32
    src3 = edge_index[0].astype(jnp.int32).reshape(NCHK, CH)
    dst1 = edge_index[1].astype(jnp.int32)
    dst3 = dst1.reshape(NCHK, CH)
    batch_c = batch.astype(jnp.int32).reshape(N, 1)

    deg_p = _sc_degree(dst1)
    deg0 = deg_p[0, :N].reshape(N, 1)
    deg1 = deg_p[1, :N].reshape(N, 1)

    h1p, dis = _tc_a(x, W1, deg0, deg1)
    s1 = _sc_edge_pass(h1p, src3, dst3)
    h2p = _tc_b(s1[0], s1[1], h1p, dis, W2, b1.reshape(1, DH))
    s2 = _sc_edge_pass(h2p, src3, dst3)

    Wfc2p = jnp.zeros((DH, 128), f32).at[:, 0:1].set(Wfc2)
    out = _tc_c(s2[0], s2[1], h2p, dis, b2.reshape(1, DH), batch_c,
                Wfc1, bfc1.reshape(1, DH), Wfc2p, bfc2.reshape(1, 1))
    return out


# speedup vs baseline: 1.0394x; 1.0390x over previous
import functools

import jax
import jax.numpy as jnp
from jax import lax
from jax.experimental import pallas as pl
from jax.experimental.pallas import tpu as pltpu
from jax.experimental.pallas import tpu_sc as plsc

N = 10000
E = 320000
D_IN = 128
DH = 64
G = 64

NC = 2
NS = 16
NW = NC * NS
L = 16

RB = 400
NB = N // RB
SL = N // NS
ZR = SL // 5
NPADH = 10240
SLH = NPADH // NS
CH = 128
NCHK = E // CH
KMAX = NCHK // NW + 1
NBIG = NW - (KMAX * NW - NCHK)

_mesh = plsc.VectorSubcoreMesh(core_axis_name="c", subcore_axis_name="s")
_sc_params = pltpu.CompilerParams(needs_layout_passes=False,
                                  use_tc_tiling_on_sc=False,
                                  skip_device_barrier=True)


def _tile_range(wid):
    base = (KMAX - 1) * wid + jnp.minimum(wid, NBIG)
    cnt = jnp.where(wid < NBIG, KMAX, KMAX - 1)
    return base, cnt


@functools.partial(
    pl.kernel,
    out_type=jax.ShapeDtypeStruct((NC, NPADH), jnp.float32),
    mesh=_mesh,
    compiler_params=_sc_params,
    scratch_types=[
        pltpu.VMEM((KMAX * CH,), jnp.int32),
        pltpu.VMEM((NPADH,), jnp.float32),
        pltpu.VMEM((SLH,), jnp.float32),
        pltpu.VMEM((SLH,), jnp.float32),
        pltpu.VMEM_SHARED((NS, NPADH), jnp.float32),
    ],
)
def _sc_degree(dst_hbm, deg_out, idx_v, hist, acc_v, tmp_v, hist_sh):
    cid = lax.axis_index("c")
    sid = lax.axis_index("s")
    wid = cid * NS + sid
    base, cnt = _tile_range(wid)

    @pl.when(cnt == KMAX)
    def _():
        pltpu.sync_copy(dst_hbm.at[pl.ds(base * CH, KMAX * CH)], idx_v)

    @pl.when(cnt != KMAX)
    def _():
        pltpu.sync_copy(dst_hbm.at[pl.ds(base * CH, (KMAX - 1) * CH)],
                        idx_v.at[pl.ds(0, (KMAX - 1) * CH)])

    def zb(i, _):
        hist[pl.ds(i * L, L)] = jnp.zeros((L,), jnp.float32)
        return 0

    lax.fori_loop(0, NPADH // L, zb, 0)

    ones = jnp.ones((L,), jnp.float32)

    def eb(i, _):
        ids = idx_v[pl.ds(i * L, L)]
        plsc.addupdate_scatter(hist, [ids], ones)
        return 0

    lax.fori_loop(0, cnt * (CH // L), eb, 0)

    pltpu.sync_copy(hist, hist_sh.at[sid])
    plsc.subcore_barrier()

    def za(i, _):
        acc_v[pl.ds(i * L, L)] = jnp.zeros((L,), jnp.float32)
        return 0

    lax.fori_loop(0, SLH // L, za, 0)
    for h in range(NS):
        pltpu.sync_copy(hist_sh.at[h, pl.ds(sid * SLH, SLH)], tmp_v)

        def ab(i, _):
            sl = pl.ds(i * L, L)
            acc_v[sl] = acc_v[sl] + tmp_v[sl]
            return 0

        lax.fori_loop(0, SLH // L, ab, 0)
    pltpu.sync_copy(acc_v, deg_out.at[cid, pl.ds(sid * SLH, SLH)])


@functools.partial(
    pl.kernel,
    out_type=jax.ShapeDtypeStruct((NC, N, DH), jnp.float32),
    mesh=_mesh,
    compiler_params=_sc_params,
    scratch_types=[
        pltpu.VMEM((KMAX, CH), jnp.int32),
        pltpu.VMEM((KMAX, CH), jnp.int32),
        pltpu.VMEM((CH, DH), jnp.float32),
        pltpu.VMEM((CH, DH), jnp.float32),
        pltpu.VMEM((ZR, DH), jnp.float32),
        pltpu.VMEM_SHARED((N, DH), jnp.float32),
        pltpu.SemaphoreType.DMA,
        pltpu.SemaphoreType.DMA,
    ],
)
def _sc_edge_pass(h_hbm, src_hbm, dst_hbm, out_hbm,
                  src_v, dst_v, rows0, rows1, zbuf, acc_sh, sem0, sem1):
    cid = lax.axis_index("c")
    sid = lax.axis_index("s")
    wid = cid * NS + sid
    base, cnt = _tile_range(wid)

    @pl.when(cnt == KMAX)
    def _():
        pltpu.sync_copy(src_hbm.at[pl.ds(base, KMAX)], src_v)
        pltpu.sync_copy(dst_hbm.at[pl.ds(base, KMAX)], dst_v)

    @pl.when(cnt != KMAX)
    def _():
        pltpu.sync_copy(src_hbm.at[pl.ds(base, KMAX - 1)],
                        src_v.at[pl.ds(0, KMAX - 1)])
        pltpu.sync_copy(dst_hbm.at[pl.ds(base, KMAX - 1)],
                        dst_v.at[pl.ds(0, KMAX - 1)])

    def zv(i, _):
        zbuf[i, pl.ds(0, L)] = jnp.zeros((L,), jnp.float32)
        zbuf[i, pl.ds(L, L)] = jnp.zeros((L,), jnp.float32)
        zbuf[i, pl.ds(2 * L, L)] = jnp.zeros((L,), jnp.float32)
        zbuf[i, pl.ds(3 * L, L)] = jnp.zeros((L,), jnp.float32)
        return 0

    lax.fori_loop(0, ZR, zv, 0)
    for z in range(SL // ZR):
        pltpu.sync_copy(zbuf, acc_sh.at[pl.ds(sid * SL + z * ZR, ZR)])
    plsc.subcore_barrier()

    pltpu.async_copy(h_hbm.at[src_v.at[0]], rows0, sem0)
    pltpu.async_copy(h_hbm.at[src_v.at[1]], rows1, sem1)

    def body(t, _):
        j0 = 2 * t
        j1 = j0 + 1
        pltpu.make_async_copy(h_hbm.at[src_v.at[j0]], rows0, sem0).wait()
        pltpu.sync_copy(rows0, acc_sh.at[dst_v.at[j0]], add=True)

        @pl.when(j0 + 2 < cnt)
        def _():
            pltpu.async_copy(h_hbm.at[src_v.at[j0 + 2]], rows0, sem0)

        pltpu.make_async_copy(h_hbm.at[src_v.at[j1]], rows1, sem1).wait()
        pltpu.sync_copy(rows1, acc_sh.at[dst_v.at[j1]], add=True)

        @pl.when(j1 + 2 < cnt)
        def _():
            pltpu.async_copy(h_hbm.at[src_v.at[j1 + 2]], rows1, sem1)

        return 0

    lax.fori_loop(0, cnt // 2, body, 0)

    @pl.when(cnt % 2 == 1)
    def _():
        j = cnt - 1
        pltpu.make_async_copy(h_hbm.at[src_v.at[j]], rows0, sem0).wait()
        pltpu.sync_copy(rows0, acc_sh.at[dst_v.at[j]], add=True)

    plsc.subcore_barrier()
    nsl = pl.ds(sid * SL, SL)
    pltpu.sync_copy(acc_sh.at[nsl], out_hbm.at[cid, nsl])


def _lane_to_col(row_major):
    ident = jnp.where(
        lax.broadcasted_iota(jnp.int32, (RB, RB), 0)
        == lax.broadcasted_iota(jnp.int32, (RB, RB), 1), 1.0, 0.0)
    return lax.dot_general(ident, row_major, (((1,), (1,)), ((), ())),
                           preferred_element_type=jnp.float32)


def _tc_a_body(x_ref, w1_ref, deg_ref, h_ref, dis_ref):
    degT = _lane_to_col(deg_ref[0])
    deg = degT[:, 0:1] + degT[:, 1:2] + 1.0
    dis = lax.rsqrt(deg)
    h = jnp.dot(x_ref[...], w1_ref[...], preferred_element_type=jnp.float32)
    h_ref[...] = h * dis
    dis_ref[...] = jnp.broadcast_to(dis, (RB, DH))


def _tc_a(x, W1, degf3):
    return pl.pallas_call(
        _tc_a_body,
        grid=(NB,),
        in_specs=[
            pl.BlockSpec((RB, D_IN), lambda i: (i, 0)),
            pl.BlockSpec((D_IN, DH), lambda i: (0, 0)),
            pl.BlockSpec((1, NC, RB), lambda i: (i, 0, 0)),
        ],
        out_specs=[
            pl.BlockSpec((RB, DH), lambda i: (i, 0)),
            pl.BlockSpec((RB, DH), lambda i: (i, 0)),
        ],
        out_shape=[
            jax.ShapeDtypeStruct((N, DH), jnp.float32),
            jax.ShapeDtypeStruct((N, DH), jnp.float32),
        ],
    )(x, W1, degf3)


def _tc_b_body(s0_ref, s1_ref, hp_ref, dis_ref, w2_ref, b1_ref, out_ref):
    dis = dis_ref[...]
    z = jax.nn.relu(dis * (s0_ref[...] + s1_ref[...] + hp_ref[...])
                    + b1_ref[...])
    out_ref[...] = jnp.dot(z, w2_ref[...],
                           preferred_element_type=jnp.float32) * dis


def _tc_b(s0, s1, h1p, dis, W2, b1r):
    return pl.pallas_call(
        _tc_b_body,
        grid=(NB,),
        in_specs=[
            pl.BlockSpec((RB, DH), lambda i: (i, 0)),
            pl.BlockSpec((RB, DH), lambda i: (i, 0)),
            pl.BlockSpec((RB, DH), lambda i: (i, 0)),
            pl.BlockSpec((RB, DH), lambda i: (i, 0)),
            pl.BlockSpec((DH, DH), lambda i: (0, 0)),
            pl.BlockSpec((1, DH), lambda i: (0, 0)),
        ],
        out_specs=pl.BlockSpec((RB, DH), lambda i: (i, 0)),
        out_shape=jax.ShapeDtypeStruct((N, DH), jnp.float32),
    )(s0, s1, h1p, dis, W2, b1r)


def _tc_c_body(s0_ref, s1_ref, hp_ref, dis_ref, b2_ref, bt_ref,
               wf1_ref, bf1_ref, wf2_ref, bf2_ref, out_ref,
               sum_acc, max_acc, cnt_acc):
    pid = pl.program_id(0)

    @pl.when(pid == 0)
    def _():
        sum_acc[...] = jnp.zeros((G, DH), jnp.float32)
        max_acc[...] = jnp.full((G, DH), -jnp.inf, jnp.float32)
        cnt_acc[...] = jnp.zeros((G, 1), jnp.float32)

    h2 = (dis_ref[...] * (s0_ref[...] + s1_ref[...] + hp_ref[...])
          + b2_ref[...])
    bt = _lane_to_col(bt_ref[0])
    gi = lax.broadcasted_iota(jnp.int32, (RB, G), 1).astype(jnp.float32)
    onehot = jnp.where(bt == gi, 1.0, 0.0)
    sum_acc[...] += lax.dot_general(
        onehot, h2, (((0,), (0,)), ((), ())),
        preferred_element_type=jnp.float32)
    cnt_acc[...] += lax.dot_general(
        onehot, jnp.ones((RB, 1), jnp.float32), (((0,), (0,)), ((), ())),
        preferred_element_type=jnp.float32)

    glo = jnp.min(bt).astype(jnp.int32)
    ghi = jnp.max(bt).astype(jnp.int32)

    def gbody(g, _):
        m = jnp.where(bt == g.astype(jnp.float32), h2, -jnp.inf)
        colmax = jnp.max(m, axis=0, keepdims=True)
        sl = pl.ds(g, 1)
        max_acc[sl, :] = jnp.maximum(max_acc[sl, :], colmax)
        return 0

    lax.fori_loop(glo, ghi + 1, gbody, 0)

    @pl.when(pid == NB - 1)
    def _():
        mean = sum_acc[...] / jnp.maximum(cnt_acc[...], 1.0)
        pooled = jnp.concatenate([mean, max_acc[...]], axis=1)
        z = jax.nn.relu(
            jnp.dot(pooled, wf1_ref[...], preferred_element_type=jnp.float32)
            + bf1_ref[...])
        o = jnp.dot(z, wf2_ref[...], preferred_element_type=jnp.float32)
        out_ref[...] = o[:, 0:1] + bf2_ref[...]


def _tc_c(s0, s1, h2p, dis, b2r, batch_c, Wfc1, bfc1r, Wfc2p, bfc2r):
    return pl.pallas_call(
        _tc_c_body,
        grid=(NB,),
        in_specs=[
            pl.BlockSpec((RB, DH), lambda i: (i, 0)),
            pl.BlockSpec((RB, DH), lambda i: (i, 0)),
            pl.BlockSpec((RB, DH), lambda i: (i, 0)),
            pl.BlockSpec((RB, DH), lambda i: (i, 0)),
            pl.BlockSpec((1, DH), lambda i: (0, 0)),
            pl.BlockSpec((1, 1, RB), lambda i: (i, 0, 0)),
            pl.BlockSpec((2 * DH, DH), lambda i: (0, 0)),
            pl.BlockSpec((1, DH), lambda i: (0, 0)),
            pl.BlockSpec((DH, 128), lambda i: (0, 0)),
            pl.BlockSpec((1, 1), lambda i: (0, 0)),
        ],
        out_specs=pl.BlockSpec((G, 1), lambda i: (0, 0)),
        out_shape=jax.ShapeDtypeStruct((G, 1), jnp.float32),
        scratch_shapes=[
            pltpu.VMEM((G, DH), jnp.float32),
            pltpu.VMEM((G, DH), jnp.float32),
            pltpu.VMEM((G, 1), jnp.float32),
        ],
    )(s0, s1, h2p, dis, b2r, batch_c, Wfc1, bfc1r, Wfc2p, bfc2r)


@jax.jit
def kernel(x, edge_index, batch, W1, b1, W2, b2, Wfc1, bfc1, Wfc2, bfc2):
    f32 = jnp.float32
    src3 = edge_index[0].astype(jnp.int32).reshape(NCHK, CH)
    dst1 = edge_index[1].astype(jnp.int32)
    dst3 = dst1.reshape(NCHK, CH)
    batch_f3 = batch.astype(f32).reshape(NB, 1, RB)

    deg_p = _sc_degree(dst1)
    degf3 = deg_p[:, :N].reshape(NC, NB, RB).transpose(1, 0, 2)

    h1p, dis = _tc_a(x, W1, degf3)
    s1 = _sc_edge_pass(h1p, src3, dst3)
    h2p = _tc_b(s1[0], s1[1], h1p, dis, W2, b1.reshape(1, DH))
    s2 = _sc_edge_pass(h2p, src3, dst3)

    Wfc2p = jnp.zeros((DH, 128), f32).at[:, 0:1].set(Wfc2)
    out = _tc_c(s2[0], s2[1], h2p, dis, b2.reshape(1, DH), batch_f3,
                Wfc1, bfc1.reshape(1, DH), Wfc2p, bfc2.reshape(1, 1))
    return out


# speedup vs baseline: 1.0699x; 1.0294x over previous
import functools

import jax
import jax.numpy as jnp
from jax import lax
from jax.experimental import pallas as pl
from jax.experimental.pallas import tpu as pltpu
from jax.experimental.pallas import tpu_sc as plsc

N = 10000
E = 320000
D_IN = 128
DH = 64
G = 64

NC = 2
NS = 16
NW = NC * NS
L = 16

RB = 400
NB = N // RB
SL = N // NS
ZR = SL // 5
NPADH = 10240
SLH = NPADH // NS
CH = 128
NCHK = E // CH
KMAX = NCHK // NW + 1
NBIG = NW - (KMAX * NW - NCHK)

_mesh = plsc.VectorSubcoreMesh(core_axis_name="c", subcore_axis_name="s")
_sc_params = pltpu.CompilerParams(needs_layout_passes=False,
                                  use_tc_tiling_on_sc=False,
                                  skip_device_barrier=True)


def _tile_range(wid):
    base = (KMAX - 1) * wid + jnp.minimum(wid, NBIG)
    cnt = jnp.where(wid < NBIG, KMAX, KMAX - 1)
    return base, cnt


@functools.partial(
    pl.kernel,
    out_type=jax.ShapeDtypeStruct((NC, NPADH), jnp.float32),
    mesh=_mesh,
    compiler_params=_sc_params,
    scratch_types=[
        pltpu.VMEM((KMAX * CH,), jnp.int32),
        pltpu.VMEM((NPADH,), jnp.float32),
        pltpu.VMEM((SLH,), jnp.float32),
        pltpu.VMEM((SLH,), jnp.float32),
        pltpu.VMEM_SHARED((NS, NPADH), jnp.float32),
    ],
)
def _sc_degree(dst_hbm, deg_out, idx_v, hist, acc_v, tmp_v, hist_sh):
    cid = lax.axis_index("c")
    sid = lax.axis_index("s")
    wid = cid * NS + sid
    base, cnt = _tile_range(wid)

    @pl.when(cnt == KMAX)
    def _():
        pltpu.sync_copy(dst_hbm.at[pl.ds(base * CH, KMAX * CH)], idx_v)

    @pl.when(cnt != KMAX)
    def _():
        pltpu.sync_copy(dst_hbm.at[pl.ds(base * CH, (KMAX - 1) * CH)],
                        idx_v.at[pl.ds(0, (KMAX - 1) * CH)])

    def zb(i, _):
        hist[pl.ds(i * L, L)] = jnp.zeros((L,), jnp.float32)
        return 0

    lax.fori_loop(0, NPADH // L, zb, 0)

    ones = jnp.ones((L,), jnp.float32)

    def eb(i, _):
        ids = idx_v[pl.ds(i * L, L)]
        plsc.addupdate_scatter(hist, [ids], ones)
        return 0

    lax.fori_loop(0, cnt * (CH // L), eb, 0)

    pltpu.sync_copy(hist, hist_sh.at[sid])
    plsc.subcore_barrier()

    def za(i, _):
        acc_v[pl.ds(i * L, L)] = jnp.zeros((L,), jnp.float32)
        return 0

    lax.fori_loop(0, SLH // L, za, 0)
    for h in range(NS):
        pltpu.sync_copy(hist_sh.at[h, pl.ds(sid * SLH, SLH)], tmp_v)

        def ab(i, _):
            sl = pl.ds(i * L, L)
            acc_v[sl] = acc_v[sl] + tmp_v[sl]
            return 0

        lax.fori_loop(0, SLH // L, ab, 0)
    pltpu.sync_copy(acc_v, deg_out.at[cid, pl.ds(sid * SLH, SLH)])


@functools.partial(
    pl.kernel,
    out_type=jax.ShapeDtypeStruct((NC, N, DH), jnp.float32),
    mesh=_mesh,
    compiler_params=_sc_params,
    scratch_types=[
        pltpu.VMEM((KMAX, CH), jnp.int32),
        pltpu.VMEM((KMAX, CH), jnp.int32),
        pltpu.VMEM((CH, DH), jnp.float32),
        pltpu.VMEM((CH, DH), jnp.float32),
        pltpu.VMEM((ZR, DH), jnp.float32),
        pltpu.VMEM_SHARED((N, DH), jnp.float32),
        pltpu.SemaphoreType.DMA,
        pltpu.SemaphoreType.DMA,
    ],
)
def _sc_edge_pass(h_hbm, src_hbm, dst_hbm, out_hbm,
                  src_v, dst_v, rows0, rows1, zbuf, acc_sh, sem0, sem1):
    cid = lax.axis_index("c")
    sid = lax.axis_index("s")
    wid = cid * NS + sid
    base, cnt = _tile_range(wid)

    @pl.when(cnt == KMAX)
    def _():
        pltpu.sync_copy(src_hbm.at[pl.ds(base, KMAX)], src_v)
        pltpu.sync_copy(dst_hbm.at[pl.ds(base, KMAX)], dst_v)

    @pl.when(cnt != KMAX)
    def _():
        pltpu.sync_copy(src_hbm.at[pl.ds(base, KMAX - 1)],
                        src_v.at[pl.ds(0, KMAX - 1)])
        pltpu.sync_copy(dst_hbm.at[pl.ds(base, KMAX - 1)],
                        dst_v.at[pl.ds(0, KMAX - 1)])

    def zv(i, _):
        zbuf[i, pl.ds(0, L)] = jnp.zeros((L,), jnp.float32)
        zbuf[i, pl.ds(L, L)] = jnp.zeros((L,), jnp.float32)
        zbuf[i, pl.ds(2 * L, L)] = jnp.zeros((L,), jnp.float32)
        zbuf[i, pl.ds(3 * L, L)] = jnp.zeros((L,), jnp.float32)
        return 0

    lax.fori_loop(0, ZR, zv, 0)
    for z in range(SL // ZR):
        pltpu.sync_copy(zbuf, acc_sh.at[pl.ds(sid * SL + z * ZR, ZR)])
    plsc.subcore_barrier()

    pltpu.async_copy(h_hbm.at[src_v.at[0]], rows0, sem0)
    pltpu.async_copy(h_hbm.at[src_v.at[1]], rows1, sem1)

    def body(t, _):
        j0 = 2 * t
        j1 = j0 + 1
        pltpu.make_async_copy(h_hbm.at[src_v.at[j0]], rows0, sem0).wait()
        pltpu.sync_copy(rows0, acc_sh.at[dst_v.at[j0]], add=True)

        @pl.when(j0 + 2 < cnt)
        def _():
            pltpu.async_copy(h_hbm.at[src_v.at[j0 + 2]], rows0, sem0)

        pltpu.make_async_copy(h_hbm.at[src_v.at[j1]], rows1, sem1).wait()
        pltpu.sync_copy(rows1, acc_sh.at[dst_v.at[j1]], add=True)

        @pl.when(j1 + 2 < cnt)
        def _():
            pltpu.async_copy(h_hbm.at[src_v.at[j1 + 2]], rows1, sem1)

        return 0

    lax.fori_loop(0, cnt // 2, body, 0)

    @pl.when(cnt % 2 == 1)
    def _():
        j = cnt - 1
        pltpu.make_async_copy(h_hbm.at[src_v.at[j]], rows0, sem0).wait()
        pltpu.sync_copy(rows0, acc_sh.at[dst_v.at[j]], add=True)

    plsc.subcore_barrier()
    nsl = pl.ds(sid * SL, SL)
    pltpu.sync_copy(acc_sh.at[nsl], out_hbm.at[cid, nsl])


def _lane_to_col(row_major):
    ident = jnp.where(
        lax.broadcasted_iota(jnp.int32, (RB, RB), 0)
        == lax.broadcasted_iota(jnp.int32, (RB, RB), 1), 1.0, 0.0)
    return lax.dot_general(ident, row_major, (((1,), (1,)), ((), ())),
                           preferred_element_type=jnp.float32)


def _dis_col(deg_ref):
    degT = _lane_to_col(deg_ref[0])
    return lax.rsqrt(degT[:, 0:1] + degT[:, 1:2] + 1.0)


def _tc_a_body(x_ref, w1_ref, deg_ref, h_ref):
    dis = _dis_col(deg_ref)
    h = jnp.dot(x_ref[...], w1_ref[...], preferred_element_type=jnp.float32)
    h_ref[...] = h * dis


def _tc_a(x, W1, degf3):
    return pl.pallas_call(
        _tc_a_body,
        grid=(NB,),
        in_specs=[
            pl.BlockSpec((RB, D_IN), lambda i: (i, 0)),
            pl.BlockSpec((D_IN, DH), lambda i: (0, 0)),
            pl.BlockSpec((1, NC, RB), lambda i: (i, 0, 0)),
        ],
        out_specs=pl.BlockSpec((RB, DH), lambda i: (i, 0)),
        out_shape=jax.ShapeDtypeStruct((N, DH), jnp.float32),
    )(x, W1, degf3)


def _tc_b_body(s_ref, hp_ref, deg_ref, w2_ref, b1_ref, out_ref):
    dis = _dis_col(deg_ref)
    z = jax.nn.relu(dis * (s_ref[0] + s_ref[1] + hp_ref[...])
                    + b1_ref[...])
    out_ref[...] = jnp.dot(z, w2_ref[...],
                           preferred_element_type=jnp.float32) * dis


def _tc_b(s, h1p, degf3, W2, b1r):
    return pl.pallas_call(
        _tc_b_body,
        grid=(NB,),
        in_specs=[
            pl.BlockSpec((NC, RB, DH), lambda i: (0, i, 0)),
            pl.BlockSpec((RB, DH), lambda i: (i, 0)),
            pl.BlockSpec((1, NC, RB), lambda i: (i, 0, 0)),
            pl.BlockSpec((DH, DH), lambda i: (0, 0)),
            pl.BlockSpec((1, DH), lambda i: (0, 0)),
        ],
        out_specs=pl.BlockSpec((RB, DH), lambda i: (i, 0)),
        out_shape=jax.ShapeDtypeStruct((N, DH), jnp.float32),
    )(s, h1p, degf3, W2, b1r)


def _tc_c_body(s_ref, hp_ref, deg_ref, b2_ref, bt_ref,
               wf1_ref, bf1_ref, wf2_ref, bf2_ref, out_ref,
               sum_acc, max_acc, cnt_acc):
    pid = pl.program_id(0)

    @pl.when(pid == 0)
    def _():
        sum_acc[...] = jnp.zeros((G, DH), jnp.float32)
        max_acc[...] = jnp.full((G, DH), -jnp.inf, jnp.float32)
        cnt_acc[...] = jnp.zeros((G, 1), jnp.float32)

    dis = _dis_col(deg_ref)
    h2 = (dis * (s_ref[0] + s_ref[1] + hp_ref[...])
          + b2_ref[...])
    bt = _lane_to_col(bt_ref[0])
    gi = lax.broadcasted_iota(jnp.int32, (RB, G), 1).astype(jnp.float32)
    onehot = jnp.where(bt == gi, 1.0, 0.0)
    sum_acc[...] += lax.dot_general(
        onehot, h2, (((0,), (0,)), ((), ())),
        preferred_element_type=jnp.float32)
    cnt_acc[...] += lax.dot_general(
        onehot, jnp.ones((RB, 1), jnp.float32), (((0,), (0,)), ((), ())),
        preferred_element_type=jnp.float32)

    glo = jnp.min(bt).astype(jnp.int32)
    ghi = jnp.max(bt).astype(jnp.int32)

    def gbody(g, _):
        m = jnp.where(bt == g.astype(jnp.float32), h2, -jnp.inf)
        colmax = jnp.max(m, axis=0, keepdims=True)
        sl = pl.ds(g, 1)
        max_acc[sl, :] = jnp.maximum(max_acc[sl, :], colmax)
        return 0

    lax.fori_loop(glo, ghi + 1, gbody, 0)

    @pl.when(pid == NB - 1)
    def _():
        mean = sum_acc[...] / jnp.maximum(cnt_acc[...], 1.0)
        pooled = jnp.concatenate([mean, max_acc[...]], axis=1)
        z = jax.nn.relu(
            jnp.dot(pooled, wf1_ref[...], preferred_element_type=jnp.float32)
            + bf1_ref[...])
        o = jnp.dot(z, wf2_ref[...], preferred_element_type=jnp.float32)
        out_ref[...] = o[:, 0:1] + bf2_ref[...]


def _tc_c(s, h2p, degf3, b2r, batch_f3, Wfc1, bfc1r, Wfc2p, bfc2r):
    return pl.pallas_call(
        _tc_c_body,
        grid=(NB,),
        in_specs=[
            pl.BlockSpec((NC, RB, DH), lambda i: (0, i, 0)),
            pl.BlockSpec((RB, DH), lambda i: (i, 0)),
            pl.BlockSpec((1, NC, RB), lambda i: (i, 0, 0)),
            pl.BlockSpec((1, DH), lambda i: (0, 0)),
            pl.BlockSpec((1, 1, RB), lambda i: (i, 0, 0)),
            pl.BlockSpec((2 * DH, DH), lambda i: (0, 0)),
            pl.BlockSpec((1, DH), lambda i: (0, 0)),
            pl.BlockSpec((DH, 128), lambda i: (0, 0)),
            pl.BlockSpec((1, 1), lambda i: (0, 0)),
        ],
        out_specs=pl.BlockSpec((G, 1), lambda i: (0, 0)),
        out_shape=jax.ShapeDtypeStruct((G, 1), jnp.float32),
        scratch_shapes=[
            pltpu.VMEM((G, DH), jnp.float32),
            pltpu.VMEM((G, DH), jnp.float32),
            pltpu.VMEM((G, 1), jnp.float32),
        ],
    )(s, h2p, degf3, b2r, batch_f3, Wfc1, bfc1r, Wfc2p, bfc2r)


@jax.jit
def kernel(x, edge_index, batch, W1, b1, W2, b2, Wfc1, bfc1, Wfc2, bfc2):
    f32 = jnp.float32
    src3 = edge_index[0].astype(jnp.int32).reshape(NCHK, CH)
    dst1 = edge_index[1].astype(jnp.int32)
    dst3 = dst1.reshape(NCHK, CH)
    batch_f3 = batch.astype(f32).reshape(NB, 1, RB)

    deg_p = _sc_degree(dst1)
    degf3 = deg_p[:, :N].reshape(NC, NB, RB).transpose(1, 0, 2)

    h1p = _tc_a(x, W1, degf3)
    s1 = _sc_edge_pass(h1p, src3, dst3)
    h2p = _tc_b(s1, h1p, degf3, W2, b1.reshape(1, DH))
    s2 = _sc_edge_pass(h2p, src3, dst3)

    Wfc2p = jnp.zeros((DH, 128), f32).at[:, 0:1].set(Wfc2)
    out = _tc_c(s2, h2p, degf3, b2.reshape(1, DH), batch_f3,
                Wfc1, bfc1.reshape(1, DH), Wfc2p, bfc2.reshape(1, 1))
    return out


# speedup vs baseline: 1.1292x; 1.0554x over previous
import functools

import jax
import jax.numpy as jnp
from jax import lax
from jax.experimental import pallas as pl
from jax.experimental.pallas import tpu as pltpu
from jax.experimental.pallas import tpu_sc as plsc

N = 10000
E = 320000
D_IN = 128
DH = 64
G = 64

NC = 2
NS = 16
NW = NC * NS
L = 16

RB = 400
NB = N // RB
SL = N // NS
ZR = SL // 5
NPADH = 10240
SLH = NPADH // NS
CH = 128
NCHK = E // CH
KMAX = NCHK // NW + 1
NBIG = NW - (KMAX * NW - NCHK)

_mesh = plsc.VectorSubcoreMesh(core_axis_name="c", subcore_axis_name="s")
_sc_params = pltpu.CompilerParams(needs_layout_passes=False,
                                  use_tc_tiling_on_sc=False,
                                  skip_device_barrier=True)


def _tile_range(wid):
    base = (KMAX - 1) * wid + jnp.minimum(wid, NBIG)
    cnt = jnp.where(wid < NBIG, KMAX, KMAX - 1)
    return base, cnt


@functools.partial(
    pl.kernel,
    out_type=jax.ShapeDtypeStruct((NC, NPADH), jnp.float32),
    mesh=_mesh,
    compiler_params=_sc_params,
    scratch_types=[
        pltpu.VMEM((KMAX, NC, CH), jnp.int32),
        pltpu.VMEM((NPADH,), jnp.float32),
        pltpu.VMEM((SLH,), jnp.float32),
        pltpu.VMEM((SLH,), jnp.float32),
        pltpu.VMEM_SHARED((NS, NPADH), jnp.float32),
    ],
)
def _sc_degree(edge_hbm, deg_out, idx_v, hist, acc_v, tmp_v, hist_sh):
    cid = lax.axis_index("c")
    sid = lax.axis_index("s")
    wid = cid * NS + sid
    base, cnt = _tile_range(wid)

    @pl.when(cnt == KMAX)
    def _():
        pltpu.sync_copy(edge_hbm.at[pl.ds(base, KMAX)], idx_v)

    @pl.when(cnt != KMAX)
    def _():
        pltpu.sync_copy(edge_hbm.at[pl.ds(base, KMAX - 1)],
                        idx_v.at[pl.ds(0, KMAX - 1)])

    def zb(i, _):
        hist[pl.ds(i * L, L)] = jnp.zeros((L,), jnp.float32)
        return 0

    lax.fori_loop(0, NPADH // L, zb, 0)

    ones = jnp.ones((L,), jnp.float32)

    def eb(i, _):
        j = i // (CH // L)
        k = i % (CH // L)
        ids = idx_v[j, 1, pl.ds(k * L, L)]
        plsc.addupdate_scatter(hist, [ids], ones)
        return 0

    lax.fori_loop(0, cnt * (CH // L), eb, 0)

    pltpu.sync_copy(hist, hist_sh.at[sid])
    plsc.subcore_barrier()

    def za(i, _):
        acc_v[pl.ds(i * L, L)] = jnp.zeros((L,), jnp.float32)
        return 0

    lax.fori_loop(0, SLH // L, za, 0)
    for h in range(NS):
        pltpu.sync_copy(hist_sh.at[h, pl.ds(sid * SLH, SLH)], tmp_v)

        def ab(i, _):
            sl = pl.ds(i * L, L)
            acc_v[sl] = acc_v[sl] + tmp_v[sl]
            return 0

        lax.fori_loop(0, SLH // L, ab, 0)
    pltpu.sync_copy(acc_v, deg_out.at[cid, pl.ds(sid * SLH, SLH)])


@functools.partial(
    pl.kernel,
    out_type=jax.ShapeDtypeStruct((NC, N, DH), jnp.float32),
    mesh=_mesh,
    compiler_params=_sc_params,
    scratch_types=[
        pltpu.VMEM((KMAX, NC, CH), jnp.int32),
        pltpu.VMEM((CH, DH), jnp.float32),
        pltpu.VMEM((CH, DH), jnp.float32),
        pltpu.VMEM((ZR, DH), jnp.float32),
        pltpu.VMEM_SHARED((N, DH), jnp.float32),
        pltpu.SemaphoreType.DMA,
        pltpu.SemaphoreType.DMA,
    ],
)
def _sc_edge_pass(h_hbm, edge_hbm, out_hbm,
                  edge_v, rows0, rows1, zbuf, acc_sh, sem0, sem1):
    cid = lax.axis_index("c")
    sid = lax.axis_index("s")
    wid = cid * NS + sid
    base, cnt = _tile_range(wid)
    def src_at(j):
        return edge_v.at[j, 0]

    def dst_at(j):
        return edge_v.at[j, 1]

    @pl.when(cnt == KMAX)
    def _():
        pltpu.sync_copy(edge_hbm.at[pl.ds(base, KMAX)], edge_v)

    @pl.when(cnt != KMAX)
    def _():
        pltpu.sync_copy(edge_hbm.at[pl.ds(base, KMAX - 1)],
                        edge_v.at[pl.ds(0, KMAX - 1)])

    def zv(i, _):
        zbuf[i, pl.ds(0, L)] = jnp.zeros((L,), jnp.float32)
        zbuf[i, pl.ds(L, L)] = jnp.zeros((L,), jnp.float32)
        zbuf[i, pl.ds(2 * L, L)] = jnp.zeros((L,), jnp.float32)
        zbuf[i, pl.ds(3 * L, L)] = jnp.zeros((L,), jnp.float32)
        return 0

    lax.fori_loop(0, ZR, zv, 0)
    for z in range(SL // ZR):
        pltpu.sync_copy(zbuf, acc_sh.at[pl.ds(sid * SL + z * ZR, ZR)])
    plsc.subcore_barrier()

    pltpu.async_copy(h_hbm.at[src_at(0)], rows0, sem0)
    pltpu.async_copy(h_hbm.at[src_at(1)], rows1, sem1)

    def body(t, _):
        j0 = 2 * t
        j1 = j0 + 1
        pltpu.make_async_copy(h_hbm.at[src_at(j0)], rows0, sem0).wait()
        pltpu.sync_copy(rows0, acc_sh.at[dst_at(j0)], add=True)

        @pl.when(j0 + 2 < cnt)
        def _():
            pltpu.async_copy(h_hbm.at[src_at(j0 + 2)], rows0, sem0)

        pltpu.make_async_copy(h_hbm.at[src_at(j1)], rows1, sem1).wait()
        pltpu.sync_copy(rows1, acc_sh.at[dst_at(j1)], add=True)

        @pl.when(j1 + 2 < cnt)
        def _():
            pltpu.async_copy(h_hbm.at[src_at(j1 + 2)], rows1, sem1)

        return 0

    lax.fori_loop(0, cnt // 2, body, 0)

    @pl.when(cnt % 2 == 1)
    def _():
        j = cnt - 1
        pltpu.make_async_copy(h_hbm.at[src_at(j)], rows0, sem0).wait()
        pltpu.sync_copy(rows0, acc_sh.at[dst_at(j)], add=True)

    plsc.subcore_barrier()
    nsl = pl.ds(sid * SL, SL)
    pltpu.sync_copy(acc_sh.at[nsl], out_hbm.at[cid, nsl])


def _lane_to_col(row_major):
    ident = jnp.where(
        lax.broadcasted_iota(jnp.int32, (RB, RB), 0)
        == lax.broadcasted_iota(jnp.int32, (RB, RB), 1), 1.0, 0.0)
    return lax.dot_general(ident, row_major, (((1,), (1,)), ((), ())),
                           preferred_element_type=jnp.float32)


def _dis_col(deg_ref):
    degT = _lane_to_col(deg_ref[0])
    return lax.rsqrt(degT[:, 0:1] + degT[:, 1:2] + 1.0)


def _tc_a_body(x_ref, w1_ref, deg_ref, h_ref):
    dis = _dis_col(deg_ref)
    h = jnp.dot(x_ref[...], w1_ref[...], preferred_element_type=jnp.float32)
    h_ref[...] = h * dis


def _tc_a(x, W1, degf3):
    return pl.pallas_call(
        _tc_a_body,
        grid=(NB,),
        in_specs=[
            pl.BlockSpec((RB, D_IN), lambda i: (i, 0)),
            pl.BlockSpec((D_IN, DH), lambda i: (0, 0)),
            pl.BlockSpec((1, NC, RB), lambda i: (i, 0, 0)),
        ],
        out_specs=pl.BlockSpec((RB, DH), lambda i: (i, 0)),
        out_shape=jax.ShapeDtypeStruct((N, DH), jnp.float32),
    )(x, W1, degf3)


def _tc_b_body(s_ref, hp_ref, deg_ref, w2_ref, b1_ref, out_ref):
    dis = _dis_col(deg_ref)
    z = jax.nn.relu(dis * (s_ref[0] + s_ref[1] + hp_ref[...])
                    + b1_ref[...])
    out_ref[...] = jnp.dot(z, w2_ref[...],
                           preferred_element_type=jnp.float32) * dis


def _tc_b(s, h1p, degf3, W2, b1r):
    return pl.pallas_call(
        _tc_b_body,
        grid=(NB,),
        in_specs=[
            pl.BlockSpec((NC, RB, DH), lambda i: (0, i, 0)),
            pl.BlockSpec((RB, DH), lambda i: (i, 0)),
            pl.BlockSpec((1, NC, RB), lambda i: (i, 0, 0)),
            pl.BlockSpec((DH, DH), lambda i: (0, 0)),
            pl.BlockSpec((1, DH), lambda i: (0, 0)),
        ],
        out_specs=pl.BlockSpec((RB, DH), lambda i: (i, 0)),
        out_shape=jax.ShapeDtypeStruct((N, DH), jnp.float32),
    )(s, h1p, degf3, W2, b1r)


def _tc_c_body(s_ref, hp_ref, deg_ref, b2_ref, bt_ref,
               wf1_ref, bf1_ref, wf2_ref, bf2_ref, out_ref,
               sum_acc, max_acc, cnt_acc):
    pid = pl.program_id(0)

    @pl.when(pid == 0)
    def _():
        sum_acc[...] = jnp.zeros((G, DH), jnp.float32)
        max_acc[...] = jnp.full((G, DH), -jnp.inf, jnp.float32)
        cnt_acc[...] = jnp.zeros((G, 1), jnp.float32)

    dis = _dis_col(deg_ref)
    h2 = (dis * (s_ref[0] + s_ref[1] + hp_ref[...])
          + b2_ref[...])
    bt = _lane_to_col(bt_ref[0])
    gi = lax.broadcasted_iota(jnp.int32, (RB, G), 1).astype(jnp.float32)
    onehot = jnp.where(bt == gi, 1.0, 0.0)
    sum_acc[...] += lax.dot_general(
        onehot, h2, (((0,), (0,)), ((), ())),
        preferred_element_type=jnp.float32)
    cnt_acc[...] += lax.dot_general(
        onehot, jnp.ones((RB, 1), jnp.float32), (((0,), (0,)), ((), ())),
        preferred_element_type=jnp.float32)

    glo = jnp.min(bt).astype(jnp.int32)
    ghi = jnp.max(bt).astype(jnp.int32)

    def gbody(g, _):
        m = jnp.where(bt == g.astype(jnp.float32), h2, -jnp.inf)
        colmax = jnp.max(m, axis=0, keepdims=True)
        sl = pl.ds(g, 1)
        max_acc[sl, :] = jnp.maximum(max_acc[sl, :], colmax)
        return 0

    lax.fori_loop(glo, ghi + 1, gbody, 0)

    @pl.when(pid == NB - 1)
    def _():
        mean = sum_acc[...] / jnp.maximum(cnt_acc[...], 1.0)
        pooled = jnp.concatenate([mean, max_acc[...]], axis=1)
        z = jax.nn.relu(
            jnp.dot(pooled, wf1_ref[...], preferred_element_type=jnp.float32)
            + bf1_ref[...])
        o = jnp.dot(z, wf2_ref[...], preferred_element_type=jnp.float32)
        out_ref[...] = o[:, 0:1] + bf2_ref[...]


def _tc_c(s, h2p, degf3, b2r, batch_f3, Wfc1, bfc1r, Wfc2p, bfc2r):
    return pl.pallas_call(
        _tc_c_body,
        grid=(NB,),
        in_specs=[
            pl.BlockSpec((NC, RB, DH), lambda i: (0, i, 0)),
            pl.BlockSpec((RB, DH), lambda i: (i, 0)),
            pl.BlockSpec((1, NC, RB), lambda i: (i, 0, 0)),
            pl.BlockSpec((1, DH), lambda i: (0, 0)),
            pl.BlockSpec((1, 1, RB), lambda i: (i, 0, 0)),
            pl.BlockSpec((2 * DH, DH), lambda i: (0, 0)),
            pl.BlockSpec((1, DH), lambda i: (0, 0)),
            pl.BlockSpec((DH, 128), lambda i: (0, 0)),
            pl.BlockSpec((1, 1), lambda i: (0, 0)),
        ],
        out_specs=pl.BlockSpec((G, 1), lambda i: (0, 0)),
        out_shape=jax.ShapeDtypeStruct((G, 1), jnp.float32),
        scratch_shapes=[
            pltpu.VMEM((G, DH), jnp.float32),
            pltpu.VMEM((G, DH), jnp.float32),
            pltpu.VMEM((G, 1), jnp.float32),
        ],
    )(s, h2p, degf3, b2r, batch_f3, Wfc1, bfc1r, Wfc2p, bfc2r)


@jax.jit
def kernel(x, edge_index, batch, W1, b1, W2, b2, Wfc1, bfc1, Wfc2, bfc2):
    f32 = jnp.float32
    edge3 = edge_index.astype(jnp.int32).reshape(2, NCHK, CH).transpose(1, 0, 2)
    batch_f3 = batch.astype(f32).reshape(NB, 1, RB)

    deg_p = _sc_degree(edge3)
    degf3 = deg_p[:, :N].reshape(NC, NB, RB).transpose(1, 0, 2)

    h1p = _tc_a(x, W1, degf3)
    s1 = _sc_edge_pass(h1p, edge3)
    h2p = _tc_b(s1, h1p, degf3, W2, b1.reshape(1, DH))
    s2 = _sc_edge_pass(h2p, edge3)

    Wfc2p = jnp.zeros((DH, 128), f32).at[:, 0:1].set(Wfc2)
    out = _tc_c(s2, h2p, degf3, b2.reshape(1, DH), batch_f3,
                Wfc1, bfc1.reshape(1, DH), Wfc2p, bfc2.reshape(1, 1))
    return out
